# Initial kernel scaffold; baseline (speedup 1.0000x reference)
import jax
import jax.numpy as jnp
from jax.experimental import pallas as pl


def kernel(feats, xyz, batch, qkv_w, qkv_b, proj_w, proj_b, q_table, k_table, v_table):
    raise NotImplementedError("write your pallas kernel here")


# speedup vs baseline: 795.9472x; 795.9472x over previous
import jax
import jax.numpy as jnp
from jax.experimental import pallas as pl
from jax.experimental.pallas import tpu as pltpu

E = 384
H = 6
D = 64
SH = 2
WIN = 0.4
QUANT = 0.01
L = 40
L2 = 2 * L - 1
LD = L2 * 3
NB = 256
CW = 41
CZ = 101
CSUM = CW + CW + CZ
WA = D + CSUM + CSUM
WV = D + CSUM


def _qkv_tables_kernel(feats_ref, w_ref, b_ref, qt_ref, kt_ref,
                       qkv_ref, tq_ref, tk_ref):
    x = feats_ref[...]
    w = w_ref[...]
    qkv = jax.lax.dot_general(x, w, (((1,), (1,)), ((), ())),
                              preferred_element_type=jnp.float32)
    qkv = qkv + b_ref[...]
    qkv_ref[...] = qkv
    scale = D ** -0.5
    for h in range(H):
        qh = qkv[:, h * D:(h + 1) * D] * scale
        kh = qkv[:, E + h * D:E + (h + 1) * D]
        qt = qt_ref[h % SH]
        kt = kt_ref[h % SH]
        tq_ref[:, h, :] = jax.lax.dot_general(
            qh, qt, (((1,), (1,)), ((), ())), preferred_element_type=jnp.float32)
        tk_ref[:, h, :] = jax.lax.dot_general(
            kh, kt, (((1,), (1,)), ((), ())), preferred_element_type=jnp.float32)


def _attn_kernel(widc_ref, widr_ref, qa_ref, ka_ref, va_ref, o_ref):
    qa = qa_ref[0]
    ka = ka_ref[0]
    att = jax.lax.dot_general(qa, ka, (((1,), (1,)), ((), ())),
                              preferred_element_type=jnp.float32)
    widq = widc_ref[0, 0]
    widr = widr_ref[0]
    mask = widq == widr
    att = jnp.where(mask, att, -1e30)
    m = jnp.max(att, axis=1, keepdims=True)
    p = jnp.exp(att - m)
    s = jnp.sum(p, axis=1, keepdims=True)
    p = p / s
    va = va_ref[0]
    o_ref[0] = jnp.dot(p, va, preferred_element_type=jnp.float32)


def _combine_proj_kernel(out1_ref, w6_ref, vt_ref, pw_ref, pb_ref, o_ref):
    x = out1_ref[...]
    parts = []
    for h in range(H):
        wb = w6_ref[:, h, :]
        vt = vt_ref[h]
        parts.append(jnp.dot(wb, vt, preferred_element_type=jnp.float32))
    y = x + jnp.concatenate(parts, axis=1)
    pw = pw_ref[...]
    out = jax.lax.dot_general(y, pw, (((1,), (1,)), ((), ())),
                              preferred_element_type=jnp.float32)
    o_ref[...] = out + pb_ref[...]


def kernel(feats, xyz, batch, qkv_w, qkv_b, proj_w, proj_b,
           q_table, k_table, v_table):
    N = feats.shape[0]
    nblk = N // NB
    f32 = jnp.float32

    qt2 = q_table.reshape(LD, SH, D).transpose(1, 0, 2)
    kt2 = k_table.reshape(LD, SH, D).transpose(1, 0, 2)
    qkv, tq, tk = pl.pallas_call(
        _qkv_tables_kernel,
        grid=(nblk,),
        in_specs=[
            pl.BlockSpec((NB, E), lambda i: (i, 0)),
            pl.BlockSpec((3 * E, E), lambda i: (0, 0)),
            pl.BlockSpec((1, 3 * E), lambda i: (0, 0)),
            pl.BlockSpec((SH, LD, D), lambda i: (0, 0, 0)),
            pl.BlockSpec((SH, LD, D), lambda i: (0, 0, 0)),
        ],
        out_specs=[
            pl.BlockSpec((NB, 3 * E), lambda i: (i, 0)),
            pl.BlockSpec((NB, H, LD), lambda i: (i, 0, 0)),
            pl.BlockSpec((NB, H, LD), lambda i: (i, 0, 0)),
        ],
        out_shape=[
            jax.ShapeDtypeStruct((N, 3 * E), f32),
            jax.ShapeDtypeStruct((N, H, LD), f32),
            jax.ShapeDtypeStruct((N, H, LD), f32),
        ],
    )(feats, qkv_w, qkv_b[None], qt2, kt2)

    scale = D ** -0.5
    q_all = qkv[:, :E].reshape(N, H, D) * scale
    k_all = qkv[:, E:2 * E].reshape(N, H, D)
    v_all = qkv[:, 2 * E:].reshape(N, H, D)

    mn = xyz.min(axis=0)
    rel_xyz = xyz - mn
    batch_i = batch.astype(jnp.int32)
    quant = jnp.array([QUANT, QUANT, QUANT], dtype=f32)
    strip_ws = [
        jnp.array([WIN, WIN, 1.0], dtype=f32),
        jnp.array([WIN, 1.0, WIN], dtype=f32),
        jnp.array([1.0, WIN, WIN], dtype=f32),
    ]
    strip_C = [(CW, CW, CZ), (CW, CZ, CW), (CZ, CW, CW)]

    qaug_l, kaug_l, vaug_l, widc_l, widr_l = [], [], [], [], []
    qd_strips = []
    for s in range(3):
        ws = strip_ws[s]
        Cs = strip_C[s]
        w_idx = jnp.floor(rel_xyz / ws).astype(jnp.int32)
        wid = ((batch_i * 1024 + w_idx[:, 0]) * 1024 + w_idx[:, 1]) * 1024 \
            + w_idx[:, 2]
        qd = jnp.floor((rel_xyz % ws) / quant).astype(jnp.int32)
        qd_strips.append(qd)
        widc_l.append(wid.reshape(nblk, NB, 1))
        widr_l.append(wid.reshape(1, N))

        tq_s = tq[:, 2 * s:2 * s + 2, :]
        tk_s = tk[:, 2 * s:2 * s + 2, :]
        oh_l, S_l, T_l = [], [], []
        for d in range(3):
            C = Cs[d]
            c = jnp.arange(C, dtype=jnp.int32)
            oh_l.append((qd[:, d:d + 1] == c[None]).astype(f32))
            gq = jnp.clip(qd[:, d:d + 1] - c[None] + (L - 1), 0, L2 - 1)
            gk = jnp.clip(c[None] - qd[:, d:d + 1] + (L - 1), 0, L2 - 1)
            S_l.append(jnp.take_along_axis(
                tq_s, jnp.broadcast_to((gq * 3 + d)[:, None, :], (N, 2, C)),
                axis=2))
            T_l.append(jnp.take_along_axis(
                tk_s, jnp.broadcast_to((gk * 3 + d)[:, None, :], (N, 2, C)),
                axis=2))
        oh = jnp.concatenate(oh_l, axis=1)
        oh2 = jnp.broadcast_to(oh[:, None, :], (N, 2, CSUM))
        S = jnp.concatenate(S_l, axis=2)
        T = jnp.concatenate(T_l, axis=2)
        q_s = q_all[:, 2 * s:2 * s + 2, :]
        k_s = k_all[:, 2 * s:2 * s + 2, :]
        v_s = v_all[:, 2 * s:2 * s + 2, :]
        qaug_l.append(jnp.concatenate([q_s, S, oh2], axis=2).transpose(1, 0, 2))
        kaug_l.append(jnp.concatenate([k_s, oh2, T], axis=2).transpose(1, 0, 2))
        vaug_l.append(jnp.concatenate([v_s, oh2], axis=2).transpose(1, 0, 2))

    qaug = jnp.concatenate(qaug_l, axis=0)
    kaug = jnp.concatenate(kaug_l, axis=0)
    vaug = jnp.concatenate(vaug_l, axis=0)
    widc = jnp.stack(widc_l, axis=0)
    widr = jnp.stack(widr_l, axis=0)

    O = pl.pallas_call(
        _attn_kernel,
        grid=(H, nblk),
        in_specs=[
            pl.BlockSpec((1, 1, NB, 1), lambda h, i: (h // 2, i, 0, 0)),
            pl.BlockSpec((1, 1, N), lambda h, i: (h // 2, 0, 0)),
            pl.BlockSpec((1, NB, WA), lambda h, i: (h, i, 0)),
            pl.BlockSpec((1, N, WA), lambda h, i: (h, 0, 0)),
            pl.BlockSpec((1, N, WV), lambda h, i: (h, 0, 0)),
        ],
        out_specs=pl.BlockSpec((1, NB, WV), lambda h, i: (h, i, 0)),
        out_shape=jax.ShapeDtypeStruct((H, N, WV), f32),
    )(widc, widr, qaug, kaug, vaug)

    out1 = O[:, :, :D].transpose(1, 0, 2).reshape(N, E)

    w6_l = []
    for s in range(3):
        qd = qd_strips[s]
        Cs = strip_C[s]
        off = D
        Wd_l = []
        for d in range(3):
            C = Cs[d]
            A = O[2 * s:2 * s + 2, :, off:off + C]
            off += C
            cs = jnp.cumsum(A, axis=2)
            total = cs[:, :, -1:]
            lidx = jnp.arange(L2, dtype=jnp.int32)
            idx = qd[:, d][None, :, None] + (L - 1) - lidx[None, None, :]
            inb = (idx >= 0) & (idx <= C - 1)
            idx_c = jnp.clip(idx, 0, C - 1)
            gat = jnp.take_along_axis(
                A, jnp.broadcast_to(idx_c, (2, N, L2)), axis=2)
            Wd = jnp.where(inb, gat, 0.0)
            j0 = qd[:, d][None, :, None] + (L - 1)
            below = jnp.take_along_axis(
                cs, jnp.broadcast_to(jnp.clip(j0 - 1, 0, C - 1), (2, N, 1)),
                axis=2)
            sfx = jnp.where(j0 <= C - 1, total - below, 0.0)
            j1 = qd[:, d][None, :, None] - (L - 1)
            pfx = jnp.where(
                j1 >= 0,
                jnp.take_along_axis(
                    cs, jnp.broadcast_to(jnp.clip(j1, 0, C - 1), (2, N, 1)),
                    axis=2),
                0.0)
            Wd = jnp.concatenate([sfx, Wd[:, :, 1:L2 - 1], pfx], axis=2)
            Wd_l.append(Wd)
        Ws = jnp.stack(Wd_l, axis=3)
        w6_l.append(Ws.reshape(2, N, LD))
    w6 = jnp.concatenate(w6_l, axis=0).transpose(1, 0, 2)

    vt6 = jnp.stack([v_table.reshape(LD, SH, D)[:, h % SH, :]
                     for h in range(H)], axis=0)
    out = pl.pallas_call(
        _combine_proj_kernel,
        grid=(nblk,),
        in_specs=[
            pl.BlockSpec((NB, E), lambda i: (i, 0)),
            pl.BlockSpec((NB, H, LD), lambda i: (i, 0, 0)),
            pl.BlockSpec((H, LD, D), lambda i: (0, 0, 0)),
            pl.BlockSpec((E, E), lambda i: (0, 0)),
            pl.BlockSpec((1, E), lambda i: (0, 0)),
        ],
        out_specs=pl.BlockSpec((NB, E), lambda i: (i, 0)),
        out_shape=jax.ShapeDtypeStruct((N, E), f32),
    )(out1, w6, vt6, proj_w, proj_b[None])
    return out


# speedup vs baseline: 919.4024x; 1.1551x over previous
import jax
import jax.numpy as jnp
from jax.experimental import pallas as pl
from jax.experimental.pallas import tpu as pltpu

E = 384
H = 6
D = 64
SH = 2
WIN = 0.4
QUANT = 0.01
L = 40
L2 = 2 * L - 1
LD = L2 * 3
NB = 256
CW = 41
CZ = 101
CSUM = CW + CW + CZ
WA = D + CSUM + CSUM
WV = D + CSUM


def _qkv_tables_kernel(feats_ref, w_ref, b_ref, qt_ref, kt_ref,
                       qkv_ref, tq_ref, tk_ref):
    x = feats_ref[...]
    w = w_ref[...]
    qkv = jax.lax.dot_general(x, w, (((1,), (1,)), ((), ())),
                              preferred_element_type=jnp.float32)
    qkv = qkv + b_ref[...]
    qkv_ref[...] = qkv
    scale = D ** -0.5
    for h in range(H):
        qh = qkv[:, h * D:(h + 1) * D] * scale
        kh = qkv[:, E + h * D:E + (h + 1) * D]
        qt = qt_ref[h % SH]
        kt = kt_ref[h % SH]
        tq_ref[:, h, :] = jax.lax.dot_general(
            qh, qt, (((1,), (1,)), ((), ())), preferred_element_type=jnp.float32)
        tk_ref[:, h, :] = jax.lax.dot_general(
            kh, kt, (((1,), (1,)), ((), ())), preferred_element_type=jnp.float32)


def _attn_kernel(widc_ref, widr_ref, qa_ref, ka_ref, va_ref, o_ref):
    qa = qa_ref[0]
    ka = ka_ref[0]
    att = jax.lax.dot_general(qa, ka, (((1,), (1,)), ((), ())),
                              preferred_element_type=jnp.float32)
    widq = widc_ref[0, 0]
    widr = widr_ref[0]
    mask = widq == widr
    att = jnp.where(mask, att, -1e30)
    m = jnp.max(att, axis=1, keepdims=True)
    p = jnp.exp(att - m)
    s = jnp.sum(p, axis=1, keepdims=True)
    p = p / s
    va = va_ref[0]
    o_ref[0] = jnp.dot(p, va, preferred_element_type=jnp.float32)


def _combine_proj_kernel(out1_ref, w6_ref, vt_ref, pw_ref, pb_ref, o_ref):
    x = out1_ref[...]
    parts = []
    for h in range(H):
        wb = w6_ref[:, h, :]
        vt = vt_ref[h]
        parts.append(jnp.dot(wb, vt, preferred_element_type=jnp.float32))
    y = x + jnp.concatenate(parts, axis=1)
    pw = pw_ref[...]
    out = jax.lax.dot_general(y, pw, (((1,), (1,)), ((), ())),
                              preferred_element_type=jnp.float32)
    o_ref[...] = out + pb_ref[...]


def kernel(feats, xyz, batch, qkv_w, qkv_b, proj_w, proj_b,
           q_table, k_table, v_table):
    N = feats.shape[0]
    nblk = N // NB
    f32 = jnp.float32

    qt2 = q_table.reshape(LD, SH, D).transpose(1, 0, 2)
    kt2 = k_table.reshape(LD, SH, D).transpose(1, 0, 2)
    qkv, tq, tk = pl.pallas_call(
        _qkv_tables_kernel,
        grid=(nblk,),
        in_specs=[
            pl.BlockSpec((NB, E), lambda i: (i, 0)),
            pl.BlockSpec((3 * E, E), lambda i: (0, 0)),
            pl.BlockSpec((1, 3 * E), lambda i: (0, 0)),
            pl.BlockSpec((SH, LD, D), lambda i: (0, 0, 0)),
            pl.BlockSpec((SH, LD, D), lambda i: (0, 0, 0)),
        ],
        out_specs=[
            pl.BlockSpec((NB, 3 * E), lambda i: (i, 0)),
            pl.BlockSpec((NB, H, LD), lambda i: (i, 0, 0)),
            pl.BlockSpec((NB, H, LD), lambda i: (i, 0, 0)),
        ],
        out_shape=[
            jax.ShapeDtypeStruct((N, 3 * E), f32),
            jax.ShapeDtypeStruct((N, H, LD), f32),
            jax.ShapeDtypeStruct((N, H, LD), f32),
        ],
    )(feats, qkv_w, qkv_b[None], qt2, kt2)

    scale = D ** -0.5
    q_all = qkv[:, :E].reshape(N, H, D) * scale
    k_all = qkv[:, E:2 * E].reshape(N, H, D)
    v_all = qkv[:, 2 * E:].reshape(N, H, D)

    mn = xyz.min(axis=0)
    rel_xyz = xyz - mn
    batch_i = batch.astype(jnp.int32)
    quant = jnp.array([QUANT, QUANT, QUANT], dtype=f32)
    strip_ws = [
        jnp.array([WIN, WIN, 1.0], dtype=f32),
        jnp.array([WIN, 1.0, WIN], dtype=f32),
        jnp.array([1.0, WIN, WIN], dtype=f32),
    ]
    strip_C = [(CW, CW, CZ), (CW, CZ, CW), (CZ, CW, CW)]

    qaug_l, kaug_l, vaug_l, widc_l, widr_l = [], [], [], [], []
    qd_strips = []
    for s in range(3):
        ws = strip_ws[s]
        Cs = strip_C[s]
        w_idx = jnp.floor(rel_xyz / ws).astype(jnp.int32)
        wid = ((batch_i * 1024 + w_idx[:, 0]) * 1024 + w_idx[:, 1]) * 1024 \
            + w_idx[:, 2]
        qd = jnp.floor((rel_xyz % ws) / quant).astype(jnp.int32)
        qd_strips.append(qd)
        widc_l.append(wid.reshape(nblk, NB, 1))
        widr_l.append(wid.reshape(1, N))

        tq_s = tq[:, 2 * s:2 * s + 2, :]
        tk_s = tk[:, 2 * s:2 * s + 2, :]
        oh_l, gq_l, gk_l = [], [], []
        for d in range(3):
            C = Cs[d]
            c = jnp.arange(C, dtype=jnp.int32)
            oh_l.append((qd[:, d:d + 1] == c[None]).astype(f32))
            gq = jnp.clip(qd[:, d:d + 1] - c[None] + (L - 1), 0, L2 - 1)
            gk = jnp.clip(c[None] - qd[:, d:d + 1] + (L - 1), 0, L2 - 1)
            gq_l.append(gq * 3 + d)
            gk_l.append(gk * 3 + d)
        gq_all = jnp.broadcast_to(
            jnp.concatenate(gq_l, axis=1)[:, None, :], (N, 2, CSUM))
        gk_all = jnp.broadcast_to(
            jnp.concatenate(gk_l, axis=1)[:, None, :], (N, 2, CSUM))
        S = jnp.take_along_axis(tq_s, gq_all, axis=2)
        T = jnp.take_along_axis(tk_s, gk_all, axis=2)
        oh = jnp.concatenate(oh_l, axis=1)
        oh2 = jnp.broadcast_to(oh[:, None, :], (N, 2, CSUM))
        q_s = q_all[:, 2 * s:2 * s + 2, :]
        k_s = k_all[:, 2 * s:2 * s + 2, :]
        v_s = v_all[:, 2 * s:2 * s + 2, :]
        qaug_l.append(jnp.concatenate([q_s, S, oh2], axis=2).transpose(1, 0, 2))
        kaug_l.append(jnp.concatenate([k_s, oh2, T], axis=2).transpose(1, 0, 2))
        vaug_l.append(jnp.concatenate([v_s, oh2], axis=2).transpose(1, 0, 2))

    qaug = jnp.concatenate(qaug_l, axis=0)
    kaug = jnp.concatenate(kaug_l, axis=0)
    vaug = jnp.concatenate(vaug_l, axis=0)
    widc = jnp.stack(widc_l, axis=0)
    widr = jnp.stack(widr_l, axis=0)

    O = pl.pallas_call(
        _attn_kernel,
        grid=(H, nblk),
        in_specs=[
            pl.BlockSpec((1, 1, NB, 1), lambda h, i: (h // 2, i, 0, 0)),
            pl.BlockSpec((1, 1, N), lambda h, i: (h // 2, 0, 0)),
            pl.BlockSpec((1, NB, WA), lambda h, i: (h, i, 0)),
            pl.BlockSpec((1, N, WA), lambda h, i: (h, 0, 0)),
            pl.BlockSpec((1, N, WV), lambda h, i: (h, 0, 0)),
        ],
        out_specs=pl.BlockSpec((1, NB, WV), lambda h, i: (h, i, 0)),
        out_shape=jax.ShapeDtypeStruct((H, N, WV), f32),
    )(widc, widr, qaug, kaug, vaug)

    out1 = O[:, :, :D].transpose(1, 0, 2).reshape(N, E)

    w6_l = []
    lidx = jnp.arange(L2, dtype=jnp.int32)
    for s in range(3):
        qd = qd_strips[s]
        Cs = strip_C[s]
        offs = [D, D + Cs[0], D + Cs[0] + Cs[1]]
        Aall = O[2 * s:2 * s + 2, :, :]
        cs = jnp.cumsum(Aall[:, :, D:], axis=2)
        int_idx_l, int_inb_l, bnd_idx_l = [], [], []
        for d in range(3):
            C = Cs[d]
            off = offs[d]
            idx = qd[:, d][:, None] + (L - 1) - lidx[None, :]
            int_inb_l.append((idx >= 0) & (idx <= C - 1))
            int_idx_l.append(off + jnp.clip(idx, 0, C - 1))
            j0 = qd[:, d][:, None] + (L - 1)
            j1 = qd[:, d][:, None] - (L - 1)
            bnd_idx_l.append(off - D + jnp.clip(j0 - 1, 0, C - 1))
            bnd_idx_l.append(off - D + jnp.clip(j1, 0, C - 1))
        int_idx = jnp.concatenate(int_idx_l, axis=1)
        gat = jnp.take_along_axis(
            Aall, jnp.broadcast_to(int_idx[None], (2, N, LD)), axis=2)
        bnd_idx = jnp.concatenate(bnd_idx_l, axis=1)
        bnd = jnp.take_along_axis(
            cs, jnp.broadcast_to(bnd_idx[None], (2, N, 6)), axis=2)
        Wd_l = []
        for d in range(3):
            C = Cs[d]
            off = offs[d] - D
            Wd = jnp.where(int_inb_l[d][None],
                           gat[:, :, d * L2:(d + 1) * L2], 0.0)
            j0 = qd[:, d][None, :, None] + (L - 1)
            j1 = qd[:, d][None, :, None] - (L - 1)
            cs_end = cs[:, :, off + C - 1][..., None]
            base = cs[:, :, off - 1][..., None] if off > 0 else 0.0
            sfx = jnp.where(j0 <= C - 1, cs_end - bnd[:, :, 2 * d][..., None],
                            0.0)
            pfx = jnp.where(j1 >= 0, bnd[:, :, 2 * d + 1][..., None] - base,
                            0.0)
            Wd_l.append(jnp.concatenate([sfx, Wd[:, :, 1:L2 - 1], pfx],
                                        axis=2))
        Ws = jnp.stack(Wd_l, axis=3)
        w6_l.append(Ws.reshape(2, N, LD))
    w6 = jnp.concatenate(w6_l, axis=0).transpose(1, 0, 2)

    vt6 = jnp.stack([v_table.reshape(LD, SH, D)[:, h % SH, :]
                     for h in range(H)], axis=0)
    out = pl.pallas_call(
        _combine_proj_kernel,
        grid=(nblk,),
        in_specs=[
            pl.BlockSpec((NB, E), lambda i: (i, 0)),
            pl.BlockSpec((NB, H, LD), lambda i: (i, 0, 0)),
            pl.BlockSpec((H, LD, D), lambda i: (0, 0, 0)),
            pl.BlockSpec((E, E), lambda i: (0, 0)),
            pl.BlockSpec((1, E), lambda i: (0, 0)),
        ],
        out_specs=pl.BlockSpec((NB, E), lambda i: (i, 0)),
        out_shape=jax.ShapeDtypeStruct((N, E), f32),
    )(out1, w6, vt6, proj_w, proj_b[None])
    return out


# speedup vs baseline: 3188.2313x; 3.4677x over previous
import jax
import jax.numpy as jnp
from jax.experimental import pallas as pl
from jax.experimental.pallas import tpu as pltpu

E = 384
H = 6
D = 64
SH = 2
WIN = 0.4
QUANT = 0.01
L = 40
L2 = 2 * L - 1
LD = L2 * 3
NB = 256
CW = 41
CZ = 101
CSUM = CW + CW + CZ
WA = D + CSUM + CSUM
WV = D + CSUM

_STRIP_WS = ((WIN, WIN, 1.0), (WIN, 1.0, WIN), (1.0, WIN, WIN))
_STRIP_C = ((CW, CW, CZ), (CW, CZ, CW), (CZ, CW, CW))


def _nbits(c):
    b = 0
    while (1 << b) <= c - 1:
        b += 1
    return b


def _barrel_left(x, amt, nbits):
    W = x.shape[-1]
    for k in range(nbits):
        bit = ((amt >> k) & 1) == 1
        rolled = pltpu.roll(x, W - (1 << k), axis=x.ndim - 1)
        x = jnp.where(bit, rolled, x)
    return x


def _qd_wid(relb, bat, ws):
    widx = []
    qd = []
    for d in range(3):
        x = relb[:, d:d + 1]
        w = jnp.float32(ws[d])
        widx.append(jnp.floor(x / w).astype(jnp.int32))
        qd.append(jnp.floor(jnp.mod(x, w) / jnp.float32(QUANT))
                  .astype(jnp.int32))
    wid = ((bat * 1024 + widx[0]) * 1024 + widx[1]) * 1024 + widx[2]
    return qd, wid


def _aug_kernel(feats_ref, w_ref, b_ref, qt_ref, kt_ref, rel_ref, bat_ref,
                qaug_ref, kaug_ref, vaug_ref, widc_ref):
    x = feats_ref[...]
    w = w_ref[...]
    qkv = jax.lax.dot_general(x, w, (((1,), (1,)), ((), ())),
                              preferred_element_type=jnp.float32)
    qkv = qkv + b_ref[...]
    relb = rel_ref[...]
    bat = bat_ref[...]
    scale = D ** -0.5
    for s in range(3):
        ws = _STRIP_WS[s]
        Cs = _STRIP_C[s]
        qd, wid = _qd_wid(relb, bat, ws)
        widc_ref[s, 0] = wid
        oh = []
        ohrev = []
        for d in range(3):
            C = Cs[d]
            iota = jax.lax.broadcasted_iota(jnp.int32, (NB, C), 1)
            oh.append((iota == qd[d]).astype(jnp.float32))
            ohrev.append((iota == (C - 1 - qd[d])).astype(jnp.float32))
        for hh in range(2):
            h = 2 * s + hh
            qh = qkv[:, h * D:(h + 1) * D] * scale
            kh = qkv[:, E + h * D:E + (h + 1) * D]
            vh = qkv[:, 2 * E + h * D:2 * E + (h + 1) * D]
            gS = []
            gT = []
            for d in range(3):
                C = Cs[d]
                WB = 128 if C == CW else 256
                nb = _nbits(C)
                qt = qt_ref[h % SH, d]
                kt = kt_ref[h % SH, d]
                tq_d = jax.lax.dot_general(
                    qh, qt, (((1,), (1,)), ((), ())),
                    preferred_element_type=jnp.float32)
                tk_d = jax.lax.dot_general(
                    kh, kt, (((1,), (1,)), ((), ())),
                    preferred_element_type=jnp.float32)
                lpad = C - L
                rpad = WB - lpad - L2
                B_q = jnp.concatenate(
                    [jnp.broadcast_to(tq_d[:, :1], (NB, lpad)), tq_d,
                     jnp.broadcast_to(tq_d[:, L2 - 1:], (NB, rpad))], axis=1)
                B_k = jnp.concatenate(
                    [jnp.broadcast_to(tk_d[:, :1], (NB, lpad)), tk_d,
                     jnp.broadcast_to(tk_d[:, L2 - 1:], (NB, rpad))], axis=1)
                gS.append(_barrel_left(B_q, qd[d], nb)[:, :C])
                gT.append(_barrel_left(B_k, (C - 1) - qd[d], nb)[:, :C])
            qaug_ref[h] = jnp.concatenate(
                [qh, gS[0], gS[1], gS[2], oh[0], oh[1], oh[2]], axis=1)
            kaug_ref[h] = jnp.concatenate(
                [kh, ohrev[0], ohrev[1], ohrev[2], gT[0], gT[1], gT[2]],
                axis=1)
            vaug_ref[h] = jnp.concatenate(
                [vh, oh[0], oh[1], oh[2]], axis=1)


def _attn_kernel(widc_ref, widr_ref, qa_ref, ka_ref, va_ref, o_ref):
    qa = qa_ref[0]
    ka = ka_ref[0]
    att = jax.lax.dot_general(qa, ka, (((1,), (1,)), ((), ())),
                              preferred_element_type=jnp.float32)
    widq = widc_ref[0, 0]
    widr = widr_ref[0]
    mask = widq == widr
    att = jnp.where(mask, att, -1e30)
    m = jnp.max(att, axis=1, keepdims=True)
    p = jnp.exp(att - m)
    s = jnp.sum(p, axis=1, keepdims=True)
    p = p / s
    o_ref[0] = jnp.dot(p, va_ref[0], preferred_element_type=jnp.float32)


def _combine_proj_kernel(o_ref, rel_ref, bat_ref, vt_ref, pw_ref, pb_ref,
                         out_ref):
    relb = rel_ref[...]
    bat = bat_ref[...]
    parts = []
    for s in range(3):
        ws = _STRIP_WS[s]
        Cs = _STRIP_C[s]
        qd, _ = _qd_wid(relb, bat, ws)
        off = D
        Wrev = []
        for d in range(3):
            C = Cs[d]
            WB = 128 if C == CW else 256
            nb = _nbits(C)
            A2 = o_ref[2 * s:2 * s + 2, :, off:off + C]
            off += C
            Apad = jnp.concatenate(
                [jnp.zeros((2, NB, L - 1), jnp.float32), A2,
                 jnp.zeros((2, NB, WB - (L - 1) - C), jnp.float32)], axis=2)
            amt = qd[d][None]
            Wint = _barrel_left(Apad, amt, nb)[:, :, :L2]
            iota = jax.lax.broadcasted_iota(jnp.int32, (2, NB, C), 2)
            sfx = jnp.sum(jnp.where(iota >= amt + (L - 1), A2, 0.0),
                          axis=2, keepdims=True)
            pfx = jnp.sum(jnp.where(iota <= amt - (L - 1), A2, 0.0),
                          axis=2, keepdims=True)
            Wrev.append(jnp.concatenate(
                [pfx, Wint[:, :, 1:L2 - 1], sfx], axis=2))
        Wcat = jnp.concatenate(Wrev, axis=2)
        for hh in range(2):
            h = 2 * s + hh
            o2 = jax.lax.dot_general(
                Wcat[hh], vt_ref[h], (((1,), (0,)), ((), ())),
                preferred_element_type=jnp.float32)
            parts.append(o_ref[h, :, :D] + o2)
    y = jnp.concatenate(parts, axis=1)
    out = jax.lax.dot_general(y, pw_ref[...], (((1,), (1,)), ((), ())),
                              preferred_element_type=jnp.float32)
    out_ref[...] = out + pb_ref[...]


def kernel(feats, xyz, batch, qkv_w, qkv_b, proj_w, proj_b,
           q_table, k_table, v_table):
    N = feats.shape[0]
    nblk = N // NB
    f32 = jnp.float32

    mn = xyz.min(axis=0)
    rel = xyz - mn
    bat_col = batch.astype(jnp.int32)[:, None]
    qt3 = q_table.transpose(2, 1, 0, 3)
    kt3 = k_table.transpose(2, 1, 0, 3)

    qaug, kaug, vaug, widc = pl.pallas_call(
        _aug_kernel,
        grid=(nblk,),
        in_specs=[
            pl.BlockSpec((NB, E), lambda i: (i, 0)),
            pl.BlockSpec((3 * E, E), lambda i: (0, 0)),
            pl.BlockSpec((1, 3 * E), lambda i: (0, 0)),
            pl.BlockSpec((SH, 3, L2, D), lambda i: (0, 0, 0, 0)),
            pl.BlockSpec((SH, 3, L2, D), lambda i: (0, 0, 0, 0)),
            pl.BlockSpec((NB, 3), lambda i: (i, 0)),
            pl.BlockSpec((NB, 1), lambda i: (i, 0)),
        ],
        out_specs=[
            pl.BlockSpec((H, NB, WA), lambda i: (0, i, 0)),
            pl.BlockSpec((H, NB, WA), lambda i: (0, i, 0)),
            pl.BlockSpec((H, NB, WV), lambda i: (0, i, 0)),
            pl.BlockSpec((3, 1, NB, 1), lambda i: (0, i, 0, 0)),
        ],
        out_shape=[
            jax.ShapeDtypeStruct((H, N, WA), f32),
            jax.ShapeDtypeStruct((H, N, WA), f32),
            jax.ShapeDtypeStruct((H, N, WV), f32),
            jax.ShapeDtypeStruct((3, nblk, NB, 1), jnp.int32),
        ],
    )(feats, qkv_w, qkv_b[None], qt3, kt3, rel, bat_col)

    widr_l = []
    batch_i = batch.astype(jnp.int32)
    for s in range(3):
        ws = jnp.array(_STRIP_WS[s], dtype=f32)
        w_idx = jnp.floor(rel / ws).astype(jnp.int32)
        wid = ((batch_i * 1024 + w_idx[:, 0]) * 1024 + w_idx[:, 1]) * 1024 \
            + w_idx[:, 2]
        widr_l.append(wid.reshape(1, N))
    widr = jnp.stack(widr_l, axis=0)

    O = pl.pallas_call(
        _attn_kernel,
        grid=(H, nblk),
        in_specs=[
            pl.BlockSpec((1, 1, NB, 1), lambda h, i: (h // 2, i, 0, 0)),
            pl.BlockSpec((1, 1, N), lambda h, i: (h // 2, 0, 0)),
            pl.BlockSpec((1, NB, WA), lambda h, i: (h, i, 0)),
            pl.BlockSpec((1, N, WA), lambda h, i: (h, 0, 0)),
            pl.BlockSpec((1, N, WV), lambda h, i: (h, 0, 0)),
        ],
        out_specs=pl.BlockSpec((1, NB, WV), lambda h, i: (h, i, 0)),
        out_shape=jax.ShapeDtypeStruct((H, N, WV), f32),
    )(widc, widr, qaug, kaug, vaug)

    vt_rev = v_table[::-1].transpose(1, 0, 2, 3).reshape(LD, SH, D)
    vt6 = jnp.stack([vt_rev[:, h % SH, :] for h in range(H)], axis=0)

    out = pl.pallas_call(
        _combine_proj_kernel,
        grid=(nblk,),
        in_specs=[
            pl.BlockSpec((H, NB, WV), lambda i: (0, i, 0)),
            pl.BlockSpec((NB, 3), lambda i: (i, 0)),
            pl.BlockSpec((NB, 1), lambda i: (i, 0)),
            pl.BlockSpec((H, LD, D), lambda i: (0, 0, 0)),
            pl.BlockSpec((E, E), lambda i: (0, 0)),
            pl.BlockSpec((1, E), lambda i: (0, 0)),
        ],
        out_specs=pl.BlockSpec((NB, E), lambda i: (i, 0)),
        out_shape=jax.ShapeDtypeStruct((N, E), f32),
    )(O, rel, bat_col, vt6, proj_w, proj_b[None])
    return out


# speedup vs baseline: 3551.2742x; 1.1139x over previous
import jax
import jax.numpy as jnp
from jax.experimental import pallas as pl
from jax.experimental.pallas import tpu as pltpu

E = 384
H = 6
D = 64
SH = 2
WIN = 0.4
QUANT = 0.01
L = 40
L2 = 2 * L - 1
LD = L2 * 3
NB = 256
CW = 41
CZ = 101
CSUM = CW + CW + CZ
WA = D + CSUM + CSUM
WV = D + CSUM

_STRIP_WS = ((WIN, WIN, 1.0), (WIN, 1.0, WIN), (1.0, WIN, WIN))
_STRIP_C = ((CW, CW, CZ), (CW, CZ, CW), (CZ, CW, CW))


def _nbits(c):
    b = 0
    while (1 << b) <= c - 1:
        b += 1
    return b


def _barrel_left(x, amt, nbits):
    W = x.shape[-1]
    for k in range(nbits):
        bit = ((amt >> k) & 1) == 1
        rolled = pltpu.roll(x, W - (1 << k), axis=x.ndim - 1)
        x = jnp.where(bit, rolled, x)
    return x


def _qd_wid(relb, bat, ws):
    widx = []
    qd = []
    for d in range(3):
        x = relb[:, d:d + 1]
        w = jnp.float32(ws[d])
        widx.append(jnp.floor(x / w).astype(jnp.int32))
        qd.append(jnp.floor(jnp.mod(x, w) / jnp.float32(QUANT))
                  .astype(jnp.int32))
    wid = ((bat * 1024 + widx[0]) * 1024 + widx[1]) * 1024 + widx[2]
    return qd, wid


def _aug_kernel(feats_ref, w_ref, b_ref, qt_ref, kt_ref, rel_ref, bat_ref,
                qaug_ref, kaug_ref, vaug_ref, widc_ref):
    x = feats_ref[...]
    w = w_ref[...]
    qkv = jax.lax.dot_general(x, w, (((1,), (1,)), ((), ())),
                              preferred_element_type=jnp.float32)
    qkv = qkv + b_ref[...]
    relb = rel_ref[...]
    bat = bat_ref[...]
    scale = D ** -0.5
    for s in range(3):
        ws = _STRIP_WS[s]
        Cs = _STRIP_C[s]
        qd, wid = _qd_wid(relb, bat, ws)
        widc_ref[s, 0] = wid
        oh = []
        ohrev = []
        for d in range(3):
            C = Cs[d]
            iota = jax.lax.broadcasted_iota(jnp.int32, (NB, C), 1)
            oh.append((iota == qd[d]).astype(jnp.float32))
            ohrev.append((iota == (C - 1 - qd[d])).astype(jnp.float32))
        qs_, ks_, vs_ = [], [], []
        tqd = [[None] * 3 for _ in range(2)]
        tkd = [[None] * 3 for _ in range(2)]
        for hh in range(2):
            h = 2 * s + hh
            qh = qkv[:, h * D:(h + 1) * D] * scale
            kh = qkv[:, E + h * D:E + (h + 1) * D]
            vh = qkv[:, 2 * E + h * D:2 * E + (h + 1) * D]
            qs_.append(qh)
            ks_.append(kh)
            vs_.append(vh)
            for d in range(3):
                tqd[hh][d] = jax.lax.dot_general(
                    qh, qt_ref[hh, d], (((1,), (1,)), ((), ())),
                    preferred_element_type=jnp.float32)
                tkd[hh][d] = jax.lax.dot_general(
                    kh, kt_ref[hh, d], (((1,), (1,)), ((), ())),
                    preferred_element_type=jnp.float32)
        gS = [[None] * 3 for _ in range(2)]
        gT = [[None] * 3 for _ in range(2)]
        groups = {}
        for d in range(3):
            groups.setdefault(Cs[d], []).append(d)
        for C, ds in groups.items():
            WB = 128 if C == CW else 256
            nb = _nbits(C)
            lpad = C - L
            rpad = WB - lpad - L2
            rows, amts = [], []
            for d in ds:
                for src in (tqd, tkd):
                    for hh in range(2):
                        t = src[hh][d]
                        rows.append(jnp.concatenate(
                            [jnp.broadcast_to(t[:, :1], (NB, lpad)), t,
                             jnp.broadcast_to(t[:, L2 - 1:], (NB, rpad))],
                            axis=1)[None])
                a0 = qd[d][None]
                a1 = ((C - 1) - qd[d])[None]
                amts += [a0, a0, a1, a1]
            y = _barrel_left(jnp.concatenate(rows, axis=0),
                             jnp.concatenate(amts, axis=0), nb)
            for gi, d in enumerate(ds):
                for hh in range(2):
                    gS[hh][d] = y[4 * gi + hh, :, :C]
                    gT[hh][d] = y[4 * gi + 2 + hh, :, :C]
        for hh in range(2):
            h = 2 * s + hh
            qaug_ref[h] = jnp.concatenate(
                [qs_[hh], gS[hh][0], gS[hh][1], gS[hh][2],
                 oh[0], oh[1], oh[2]], axis=1)
            kaug_ref[h] = jnp.concatenate(
                [ks_[hh], ohrev[0], ohrev[1], ohrev[2],
                 gT[hh][0], gT[hh][1], gT[hh][2]], axis=1)
            vaug_ref[h] = jnp.concatenate(
                [vs_[hh], oh[0], oh[1], oh[2]], axis=1)


def _attn_kernel(widc_ref, widr_ref, qa_ref, ka_ref, va_ref, o_ref):
    qa = qa_ref[0]
    ka = ka_ref[0]
    att = jax.lax.dot_general(qa, ka, (((1,), (1,)), ((), ())),
                              preferred_element_type=jnp.float32)
    widq = widc_ref[0, 0]
    widr = widr_ref[0]
    mask = widq == widr
    att = jnp.where(mask, att, -1e30)
    m = jnp.max(att, axis=1, keepdims=True)
    p = jnp.exp(att - m)
    s = jnp.sum(p, axis=1, keepdims=True)
    p = p / s
    o_ref[0] = jnp.dot(p, va_ref[0], preferred_element_type=jnp.float32)


def _combine_proj_kernel(o_ref, rel_ref, bat_ref, vt_ref, pw_ref, pb_ref,
                         out_ref):
    relb = rel_ref[...]
    bat = bat_ref[...]
    parts = []
    for s in range(3):
        ws = _STRIP_WS[s]
        Cs = _STRIP_C[s]
        qd, _ = _qd_wid(relb, bat, ws)
        offs = [D, D + Cs[0], D + Cs[0] + Cs[1]]
        Wrev = [None] * 3
        groups = {}
        for d in range(3):
            groups.setdefault(Cs[d], []).append(d)
        for C, ds in groups.items():
            WB = 128 if C == CW else 256
            nb = _nbits(C)
            rows, amts = [], []
            for d in ds:
                A2 = o_ref[2 * s:2 * s + 2, :, offs[d]:offs[d] + C]
                rows.append(jnp.concatenate(
                    [jnp.zeros((2, NB, L - 1), jnp.float32), A2,
                     jnp.zeros((2, NB, WB - (L - 1) - C), jnp.float32)],
                    axis=2))
                amts.append(jnp.broadcast_to(qd[d][None], (2, NB, 1)))
            y = _barrel_left(jnp.concatenate(rows, axis=0),
                             jnp.concatenate(amts, axis=0), nb)
            for gi, d in enumerate(ds):
                A2 = o_ref[2 * s:2 * s + 2, :, offs[d]:offs[d] + C]
                Wint = y[2 * gi:2 * gi + 2, :, :L2]
                amt = qd[d][None]
                iota = jax.lax.broadcasted_iota(jnp.int32, (2, NB, C), 2)
                sfx = jnp.sum(jnp.where(iota >= amt + (L - 1), A2, 0.0),
                              axis=2, keepdims=True)
                pfx = jnp.sum(jnp.where(iota <= amt - (L - 1), A2, 0.0),
                              axis=2, keepdims=True)
                Wrev[d] = jnp.concatenate(
                    [pfx, Wint[:, :, 1:L2 - 1], sfx], axis=2)
        Wcat = jnp.concatenate(Wrev, axis=2)
        for hh in range(2):
            h = 2 * s + hh
            o2 = jax.lax.dot_general(
                Wcat[hh], vt_ref[h], (((1,), (0,)), ((), ())),
                preferred_element_type=jnp.float32)
            parts.append(o_ref[h, :, :D] + o2)
    y = jnp.concatenate(parts, axis=1)
    out = jax.lax.dot_general(y, pw_ref[...], (((1,), (1,)), ((), ())),
                              preferred_element_type=jnp.float32)
    out_ref[...] = out + pb_ref[...]


def kernel(feats, xyz, batch, qkv_w, qkv_b, proj_w, proj_b,
           q_table, k_table, v_table):
    N = feats.shape[0]
    nblk = N // NB
    f32 = jnp.float32

    mn = xyz.min(axis=0)
    rel = xyz - mn
    bat_col = batch.astype(jnp.int32)[:, None]
    qt3 = q_table.transpose(2, 1, 0, 3)
    kt3 = k_table.transpose(2, 1, 0, 3)

    qaug, kaug, vaug, widc = pl.pallas_call(
        _aug_kernel,
        grid=(nblk,),
        in_specs=[
            pl.BlockSpec((NB, E), lambda i: (i, 0)),
            pl.BlockSpec((3 * E, E), lambda i: (0, 0)),
            pl.BlockSpec((1, 3 * E), lambda i: (0, 0)),
            pl.BlockSpec((SH, 3, L2, D), lambda i: (0, 0, 0, 0)),
            pl.BlockSpec((SH, 3, L2, D), lambda i: (0, 0, 0, 0)),
            pl.BlockSpec((NB, 3), lambda i: (i, 0)),
            pl.BlockSpec((NB, 1), lambda i: (i, 0)),
        ],
        out_specs=[
            pl.BlockSpec((H, NB, WA), lambda i: (0, i, 0)),
            pl.BlockSpec((H, NB, WA), lambda i: (0, i, 0)),
            pl.BlockSpec((H, NB, WV), lambda i: (0, i, 0)),
            pl.BlockSpec((3, 1, NB, 1), lambda i: (0, i, 0, 0)),
        ],
        out_shape=[
            jax.ShapeDtypeStruct((H, N, WA), f32),
            jax.ShapeDtypeStruct((H, N, WA), f32),
            jax.ShapeDtypeStruct((H, N, WV), f32),
            jax.ShapeDtypeStruct((3, nblk, NB, 1), jnp.int32),
        ],
    )(feats, qkv_w, qkv_b[None], qt3, kt3, rel, bat_col)

    widr_l = []
    batch_i = batch.astype(jnp.int32)
    for s in range(3):
        ws = jnp.array(_STRIP_WS[s], dtype=f32)
        w_idx = jnp.floor(rel / ws).astype(jnp.int32)
        wid = ((batch_i * 1024 + w_idx[:, 0]) * 1024 + w_idx[:, 1]) * 1024 \
            + w_idx[:, 2]
        widr_l.append(wid.reshape(1, N))
    widr = jnp.stack(widr_l, axis=0)

    O = pl.pallas_call(
        _attn_kernel,
        grid=(H, nblk),
        in_specs=[
            pl.BlockSpec((1, 1, NB, 1), lambda h, i: (h // 2, i, 0, 0)),
            pl.BlockSpec((1, 1, N), lambda h, i: (h // 2, 0, 0)),
            pl.BlockSpec((1, NB, WA), lambda h, i: (h, i, 0)),
            pl.BlockSpec((1, N, WA), lambda h, i: (h, 0, 0)),
            pl.BlockSpec((1, N, WV), lambda h, i: (h, 0, 0)),
        ],
        out_specs=pl.BlockSpec((1, NB, WV), lambda h, i: (h, i, 0)),
        out_shape=jax.ShapeDtypeStruct((H, N, WV), f32),
    )(widc, widr, qaug, kaug, vaug)

    vt_rev = v_table[::-1].transpose(1, 0, 2, 3).reshape(LD, SH, D)
    vt6 = jnp.stack([vt_rev[:, h % SH, :] for h in range(H)], axis=0)

    out = pl.pallas_call(
        _combine_proj_kernel,
        grid=(nblk,),
        in_specs=[
            pl.BlockSpec((H, NB, WV), lambda i: (0, i, 0)),
            pl.BlockSpec((NB, 3), lambda i: (i, 0)),
            pl.BlockSpec((NB, 1), lambda i: (i, 0)),
            pl.BlockSpec((H, LD, D), lambda i: (0, 0, 0)),
            pl.BlockSpec((E, E), lambda i: (0, 0)),
            pl.BlockSpec((1, E), lambda i: (0, 0)),
        ],
        out_specs=pl.BlockSpec((NB, E), lambda i: (i, 0)),
        out_shape=jax.ShapeDtypeStruct((N, E), f32),
    )(O, rel, bat_col, vt6, proj_w, proj_b[None])
    return out
